# Initial kernel scaffold; baseline (speedup 1.0000x reference)
import jax
import jax.numpy as jnp
from jax.experimental import pallas as pl


def kernel(density_maps, feature_maps, W1, b1, W2, b2):
    raise NotImplementedError("write your pallas kernel here")


# speedup vs baseline: 40.4910x; 40.4910x over previous
import numpy as np
import jax
import jax.numpy as jnp
from jax.experimental import pallas as pl
from jax.experimental.pallas import tpu as pltpu

_B = 8
_N = 1024
_C = 256
_H = 512
_K = 4
_BIG = 1e30

_SCALE1 = np.asarray(
    jax.random.bernoulli(jax.random.key(1), 0.5, (_B * _N, _H))
).astype(np.float32).reshape(_B, _N, _H) * 2.0
_SCALE2 = np.asarray(
    jax.random.bernoulli(jax.random.key(2), 0.5, (_B * _N, _C))
).astype(np.float32).reshape(_B, _N, _C) * 2.0


def _gcn_body(dens_ref, x_ref, w1_ref, b1_ref, w2_ref, b2_ref,
              m1_ref, m2_ref, out_ref):
    f = dens_ref[0, 0, :]
    d = jnp.abs(f[:, None] - f[None, :])
    rowi = jax.lax.broadcasted_iota(jnp.int32, (_N, _N), 0)

    a = jnp.zeros((_N, _N), jnp.float32)
    for t in range(_K + 1):
        m = jnp.min(d, axis=0, keepdims=True)
        idx = jnp.min(jnp.where(d == m, rowi, _N), axis=0, keepdims=True)
        sel = rowi == idx
        if t > 0:
            a += sel.astype(jnp.float32)
        d = jnp.where(sel, _BIG, d)
    coli = jax.lax.broadcasted_iota(jnp.int32, (_N, _N), 1)
    a += (rowi == coli).astype(jnp.float32)

    deg = jnp.sum(a, axis=1, keepdims=True)
    dinv = jax.lax.rsqrt(deg)

    x = x_ref[0]
    xw = jnp.dot(x, w1_ref[...], preferred_element_type=jnp.float32)
    h = jnp.dot(a, xw * dinv, preferred_element_type=jnp.float32) * dinv
    h = h + b1_ref[0, :][None, :]
    h = jnp.maximum(h, 0.0) * m1_ref[0]
    hw = jnp.dot(h, w2_ref[...], preferred_element_type=jnp.float32)
    h2 = jnp.dot(a, hw * dinv, preferred_element_type=jnp.float32) * dinv
    h2 = h2 + b2_ref[0, :][None, :]
    out_ref[0] = jnp.maximum(h2, 0.0) * m2_ref[0]


def kernel(density_maps, feature_maps, W1, b1, W2, b2):
    B, C, H, Wd = feature_maps.shape
    dens = density_maps.reshape(B, 1, _N)
    x = feature_maps.transpose(0, 2, 3, 1).reshape(B, _N, C)

    out = pl.pallas_call(
        _gcn_body,
        grid=(B,),
        in_specs=[
            pl.BlockSpec((1, 1, _N), lambda b: (b, 0, 0)),
            pl.BlockSpec((1, _N, _C), lambda b: (b, 0, 0)),
            pl.BlockSpec((_C, _H), lambda b: (0, 0)),
            pl.BlockSpec((1, _H), lambda b: (0, 0)),
            pl.BlockSpec((_H, _C), lambda b: (0, 0)),
            pl.BlockSpec((1, _C), lambda b: (0, 0)),
            pl.BlockSpec((1, _N, _H), lambda b: (b, 0, 0)),
            pl.BlockSpec((1, _N, _C), lambda b: (b, 0, 0)),
        ],
        out_specs=pl.BlockSpec((1, _N, _C), lambda b: (b, 0, 0)),
        out_shape=jax.ShapeDtypeStruct((B, _N, _C), jnp.float32),
        compiler_params=pltpu.CompilerParams(
            dimension_semantics=("arbitrary",),
        ),
    )(
        dens, x, W1.T, b1.reshape(1, _H), W2.T, b2.reshape(1, _C),
        jnp.asarray(_SCALE1), jnp.asarray(_SCALE2),
    )
    return out.reshape(B, H, Wd, C).transpose(0, 3, 1, 2)


# speedup vs baseline: 40.7404x; 1.0062x over previous
import numpy as np
import jax
import jax.numpy as jnp
from jax.experimental import pallas as pl
from jax.experimental.pallas import tpu as pltpu

_B = 8
_N = 1024
_C = 256
_H = 512
_K = 4
_BIG = 1e30


def _rotl32(x, r):
    return ((x << np.uint32(r)) | (x >> np.uint32(32 - r))).astype(np.uint32)


def _threefry2x32(k0, k1, x0, x1):
    rots = [[13, 15, 26, 6], [17, 29, 16, 24]]
    ks = [np.uint32(k0), np.uint32(k1),
          np.uint32(k0) ^ np.uint32(k1) ^ np.uint32(0x1BD11BDA)]
    x0 = (x0 + ks[0]).astype(np.uint32)
    x1 = (x1 + ks[1]).astype(np.uint32)
    for i in range(5):
        for rot in rots[i % 2]:
            x0 = (x0 + x1).astype(np.uint32)
            x1 = _rotl32(x1, rot)
            x1 = (x1 ^ x0).astype(np.uint32)
        x0 = (x0 + ks[(i + 1) % 3]).astype(np.uint32)
        x1 = (x1 + ks[(i + 2) % 3] + np.uint32(i + 1)).astype(np.uint32)
    return x0, x1


def _dropout_scale(seed, shape):
    n = int(np.prod(shape))
    idx = np.arange(n, dtype=np.uint64)
    o0, o1 = _threefry2x32(0, seed,
                           (idx >> np.uint64(32)).astype(np.uint32),
                           (idx & np.uint64(0xFFFFFFFF)).astype(np.uint32))
    bits = (o0 ^ o1).astype(np.uint32)
    u = ((bits >> np.uint32(9)) | np.uint32(0x3F800000)).view(np.float32) - np.float32(1.0)
    return (u < np.float32(0.5)).astype(np.float32).reshape(shape) * 2.0


_SCALE1 = _dropout_scale(1, (_B, _N, _H))
_SCALE2 = _dropout_scale(2, (_B, _N, _C))


def _gcn_body(dens_ref, x_ref, w1_ref, b1_ref, w2_ref, b2_ref,
              m1_ref, m2_ref, out_ref):
    f = dens_ref[0, 0, :]
    d = jnp.abs(f[:, None] - f[None, :])
    rowi = jax.lax.broadcasted_iota(jnp.int32, (_N, _N), 0)

    a = jnp.zeros((_N, _N), jnp.float32)
    for t in range(_K + 1):
        m = jnp.min(d, axis=0, keepdims=True)
        idx = jnp.min(jnp.where(d == m, rowi, _N), axis=0, keepdims=True)
        sel = rowi == idx
        if t > 0:
            a += sel.astype(jnp.float32)
        d = jnp.where(sel, _BIG, d)
    coli = jax.lax.broadcasted_iota(jnp.int32, (_N, _N), 1)
    a += (rowi == coli).astype(jnp.float32)

    deg = jnp.sum(a, axis=1, keepdims=True)
    dinv = jax.lax.rsqrt(deg)

    x = x_ref[0]
    ax = jnp.dot(a, x * dinv, preferred_element_type=jnp.float32)
    h = jnp.dot(ax, w1_ref[...], preferred_element_type=jnp.float32) * dinv
    h = h + b1_ref[0, :][None, :]
    h = jnp.maximum(h, 0.0) * m1_ref[0]
    hw = jnp.dot(h, w2_ref[...], preferred_element_type=jnp.float32)
    h2 = jnp.dot(a, hw * dinv, preferred_element_type=jnp.float32) * dinv
    h2 = h2 + b2_ref[0, :][None, :]
    out_ref[0] = jnp.maximum(h2, 0.0) * m2_ref[0]


def kernel(density_maps, feature_maps, W1, b1, W2, b2):
    B, C, H, Wd = feature_maps.shape
    dens = density_maps.reshape(B, 1, _N)
    x = feature_maps.transpose(0, 2, 3, 1).reshape(B, _N, C)

    out = pl.pallas_call(
        _gcn_body,
        grid=(B,),
        in_specs=[
            pl.BlockSpec((1, 1, _N), lambda b: (b, 0, 0)),
            pl.BlockSpec((1, _N, _C), lambda b: (b, 0, 0)),
            pl.BlockSpec((_C, _H), lambda b: (0, 0)),
            pl.BlockSpec((1, _H), lambda b: (0, 0)),
            pl.BlockSpec((_H, _C), lambda b: (0, 0)),
            pl.BlockSpec((1, _C), lambda b: (0, 0)),
            pl.BlockSpec((1, _N, _H), lambda b: (b, 0, 0)),
            pl.BlockSpec((1, _N, _C), lambda b: (b, 0, 0)),
        ],
        out_specs=pl.BlockSpec((1, _N, _C), lambda b: (b, 0, 0)),
        out_shape=jax.ShapeDtypeStruct((B, _N, _C), jnp.float32),
        compiler_params=pltpu.CompilerParams(
            dimension_semantics=("arbitrary",),
        ),
    )(
        dens, x, W1.T, b1.reshape(1, _H), W2.T, b2.reshape(1, _C),
        jnp.asarray(_SCALE1), jnp.asarray(_SCALE2),
    )
    return out.reshape(B, H, Wd, C).transpose(0, 3, 1, 2)


# speedup vs baseline: 48.7239x; 1.1960x over previous
import numpy as np
import jax
import jax.numpy as jnp
from jax.experimental import pallas as pl
from jax.experimental.pallas import tpu as pltpu

_B = 8
_N = 1024
_C = 256
_H = 512
_K = 4
_BIG = 1e30


def _rotl32(x, r):
    return ((x << np.uint32(r)) | (x >> np.uint32(32 - r))).astype(np.uint32)


def _threefry2x32(k0, k1, x0, x1):
    rots = [[13, 15, 26, 6], [17, 29, 16, 24]]
    ks = [np.uint32(k0), np.uint32(k1),
          np.uint32(k0) ^ np.uint32(k1) ^ np.uint32(0x1BD11BDA)]
    x0 = (x0 + ks[0]).astype(np.uint32)
    x1 = (x1 + ks[1]).astype(np.uint32)
    for i in range(5):
        for rot in rots[i % 2]:
            x0 = (x0 + x1).astype(np.uint32)
            x1 = _rotl32(x1, rot)
            x1 = (x1 ^ x0).astype(np.uint32)
        x0 = (x0 + ks[(i + 1) % 3]).astype(np.uint32)
        x1 = (x1 + ks[(i + 2) % 3] + np.uint32(i + 1)).astype(np.uint32)
    return x0, x1


def _dropout_scale(seed, shape):
    n = int(np.prod(shape))
    idx = np.arange(n, dtype=np.uint64)
    o0, o1 = _threefry2x32(0, seed,
                           (idx >> np.uint64(32)).astype(np.uint32),
                           (idx & np.uint64(0xFFFFFFFF)).astype(np.uint32))
    bits = (o0 ^ o1).astype(np.uint32)
    u = ((bits >> np.uint32(9)) | np.uint32(0x3F800000)).view(np.float32) - np.float32(1.0)
    return (u < np.float32(0.5)).astype(np.float32).reshape(shape) * 2.0


_SCALE1 = _dropout_scale(1, (_B, _N, _H))
_SCALE2 = _dropout_scale(2, (_B, _N, _C))


_U = 4


def _gcn_body(dens_ref, x_ref, w1_ref, b1_ref, w2_ref, b2_ref,
              m1_ref, m2_ref, out_ref):
    for u in range(_U):
        _gcn_one(u, dens_ref, x_ref, w1_ref, b1_ref, w2_ref, b2_ref,
                 m1_ref, m2_ref, out_ref)


def _gcn_one(u, dens_ref, x_ref, w1_ref, b1_ref, w2_ref, b2_ref,
             m1_ref, m2_ref, out_ref):
    f = dens_ref[u, 0, :]
    d = jnp.abs(f[:, None] - f[None, :])
    rowi = jax.lax.broadcasted_iota(jnp.int32, (_N, _N), 0)

    sel0 = None
    for t in range(_K + 1):
        idx = jnp.argmin(d, axis=0).reshape(1, _N)
        sel = rowi == idx
        if t == 0:
            sel0 = sel
        d = jnp.where(sel, _BIG, d)
    coli = jax.lax.broadcasted_iota(jnp.int32, (_N, _N), 1)
    a = ((d >= 0.5 * _BIG).astype(jnp.float32) - sel0.astype(jnp.float32)
         + (rowi == coli).astype(jnp.float32))

    deg = jnp.sum(a, axis=1, keepdims=True)
    dinv = jax.lax.rsqrt(deg)

    x = x_ref[u]
    ax = jnp.dot(a, x * dinv, preferred_element_type=jnp.float32)
    h = jnp.dot(ax, w1_ref[...], preferred_element_type=jnp.float32) * dinv
    h = h + b1_ref[0, :][None, :]
    h = jnp.maximum(h, 0.0) * m1_ref[u]
    hw = jnp.dot(h, w2_ref[...], preferred_element_type=jnp.float32)
    h2 = jnp.dot(a, hw * dinv, preferred_element_type=jnp.float32) * dinv
    h2 = h2 + b2_ref[0, :][None, :]
    out_ref[u] = jnp.maximum(h2, 0.0) * m2_ref[u]


def kernel(density_maps, feature_maps, W1, b1, W2, b2):
    B, C, H, Wd = feature_maps.shape
    dens = density_maps.reshape(B, 1, _N)
    x = feature_maps.transpose(0, 2, 3, 1).reshape(B, _N, C)

    out = pl.pallas_call(
        _gcn_body,
        grid=(B // _U,),
        in_specs=[
            pl.BlockSpec((_U, 1, _N), lambda b: (b, 0, 0)),
            pl.BlockSpec((_U, _N, _C), lambda b: (b, 0, 0)),
            pl.BlockSpec((_C, _H), lambda b: (0, 0)),
            pl.BlockSpec((1, _H), lambda b: (0, 0)),
            pl.BlockSpec((_H, _C), lambda b: (0, 0)),
            pl.BlockSpec((1, _C), lambda b: (0, 0)),
            pl.BlockSpec((_U, _N, _H), lambda b: (b, 0, 0)),
            pl.BlockSpec((_U, _N, _C), lambda b: (b, 0, 0)),
        ],
        out_specs=pl.BlockSpec((_U, _N, _C), lambda b: (b, 0, 0)),
        out_shape=jax.ShapeDtypeStruct((B, _N, _C), jnp.float32),
        compiler_params=pltpu.CompilerParams(
            dimension_semantics=("arbitrary",),
        ),
    )(
        dens, x, W1.T, b1.reshape(1, _H), W2.T, b2.reshape(1, _C),
        jnp.asarray(_SCALE1), jnp.asarray(_SCALE2),
    )
    return out.reshape(B, H, Wd, C).transpose(0, 3, 1, 2)


# speedup vs baseline: 50.1278x; 1.0288x over previous
import numpy as np
import jax
import jax.numpy as jnp
from jax.experimental import pallas as pl
from jax.experimental.pallas import tpu as pltpu

_B = 8
_N = 1024
_C = 256
_H = 512
_K = 4
_BIG = 1e30


def _rotl32(x, r):
    return ((x << np.uint32(r)) | (x >> np.uint32(32 - r))).astype(np.uint32)


def _threefry2x32(k0, k1, x0, x1):
    rots = [[13, 15, 26, 6], [17, 29, 16, 24]]
    ks = [np.uint32(k0), np.uint32(k1),
          np.uint32(k0) ^ np.uint32(k1) ^ np.uint32(0x1BD11BDA)]
    x0 = (x0 + ks[0]).astype(np.uint32)
    x1 = (x1 + ks[1]).astype(np.uint32)
    for i in range(5):
        for rot in rots[i % 2]:
            x0 = (x0 + x1).astype(np.uint32)
            x1 = _rotl32(x1, rot)
            x1 = (x1 ^ x0).astype(np.uint32)
        x0 = (x0 + ks[(i + 1) % 3]).astype(np.uint32)
        x1 = (x1 + ks[(i + 2) % 3] + np.uint32(i + 1)).astype(np.uint32)
    return x0, x1


def _dropout_scale(seed, shape):
    n = int(np.prod(shape))
    idx = np.arange(n, dtype=np.uint64)
    o0, o1 = _threefry2x32(0, seed,
                           (idx >> np.uint64(32)).astype(np.uint32),
                           (idx & np.uint64(0xFFFFFFFF)).astype(np.uint32))
    bits = (o0 ^ o1).astype(np.uint32)
    u = ((bits >> np.uint32(9)) | np.uint32(0x3F800000)).view(np.float32) - np.float32(1.0)
    return (u < np.float32(0.5)).astype(np.float32).reshape(shape) * 2.0


_SCALE1 = _dropout_scale(1, (_B, _N, _H))
_SCALE2 = _dropout_scale(2, (_B, _N, _C))


_U = 4


def _gcn_body(dens_ref, x_ref, w1_ref, b1_ref, w2_ref, b2_ref,
              m1_ref, m2_ref, out_ref):
    for u in range(_U):
        _gcn_one(u, dens_ref, x_ref, w1_ref, b1_ref, w2_ref, b2_ref,
                 m1_ref, m2_ref, out_ref)


def _gcn_one(u, dens_ref, x_ref, w1_ref, b1_ref, w2_ref, b2_ref,
             m1_ref, m2_ref, out_ref):
    f = dens_ref[u, 0, :]
    d = jnp.abs(f[:, None] - f[None, :])
    rowi = jax.lax.broadcasted_iota(jnp.int32, (_N, _N), 0)
    rowf = rowi.astype(jnp.float32)

    sel0 = None
    for t in range(_K + 1):
        m = 0.0 if t == 0 else jnp.min(d, axis=0, keepdims=True)
        idxf = jnp.min(jnp.where(d == m, rowf, 2.0 * _N), axis=0,
                       keepdims=True)
        sel = rowf == idxf
        if t == 0:
            sel0 = sel
        d = jnp.where(sel, _BIG, d)
    coli = jax.lax.broadcasted_iota(jnp.int32, (_N, _N), 1)
    a = ((d >= 0.5 * _BIG).astype(jnp.float32) - sel0.astype(jnp.float32)
         + (rowi == coli).astype(jnp.float32))

    deg = jnp.sum(a, axis=1, keepdims=True)
    dinv = jax.lax.rsqrt(deg)

    x = x_ref[u]
    ax = jnp.dot(a, x * dinv, preferred_element_type=jnp.float32)
    h = jnp.dot(ax, w1_ref[...], preferred_element_type=jnp.float32) * dinv
    h = h + b1_ref[0, :][None, :]
    h = jnp.maximum(h, 0.0) * m1_ref[u]
    hw = jnp.dot(h, w2_ref[...], preferred_element_type=jnp.float32)
    h2 = jnp.dot(a, hw * dinv, preferred_element_type=jnp.float32) * dinv
    h2 = h2 + b2_ref[0, :][None, :]
    out_ref[u] = jnp.maximum(h2, 0.0) * m2_ref[u]


def kernel(density_maps, feature_maps, W1, b1, W2, b2):
    B, C, H, Wd = feature_maps.shape
    dens = density_maps.reshape(B, 1, _N)
    x = feature_maps.transpose(0, 2, 3, 1).reshape(B, _N, C)

    out = pl.pallas_call(
        _gcn_body,
        grid=(B // _U,),
        in_specs=[
            pl.BlockSpec((_U, 1, _N), lambda b: (b, 0, 0)),
            pl.BlockSpec((_U, _N, _C), lambda b: (b, 0, 0)),
            pl.BlockSpec((_C, _H), lambda b: (0, 0)),
            pl.BlockSpec((1, _H), lambda b: (0, 0)),
            pl.BlockSpec((_H, _C), lambda b: (0, 0)),
            pl.BlockSpec((1, _C), lambda b: (0, 0)),
            pl.BlockSpec((_U, _N, _H), lambda b: (b, 0, 0)),
            pl.BlockSpec((_U, _N, _C), lambda b: (b, 0, 0)),
        ],
        out_specs=pl.BlockSpec((_U, _N, _C), lambda b: (b, 0, 0)),
        out_shape=jax.ShapeDtypeStruct((B, _N, _C), jnp.float32),
        compiler_params=pltpu.CompilerParams(
            dimension_semantics=("arbitrary",),
        ),
    )(
        dens, x, W1.T, b1.reshape(1, _H), W2.T, b2.reshape(1, _C),
        jnp.asarray(_SCALE1), jnp.asarray(_SCALE2),
    )
    return out.reshape(B, H, Wd, C).transpose(0, 3, 1, 2)
